# baseline (speedup 1.0000x reference)
import jax
import jax.numpy as jnp
from jax.experimental import pallas as pl
from jax.experimental.pallas import tpu as pltpu

_CD = jnp.bfloat16
_NT = (((1,), (1,)), ((), ()))


def _make_body(R, L):
    def body(*refs):
        x_ref = refs[0]
        adj_refs = refs[1:1 + R]
        rel_ref, w0_ref, b0_ref, w1_ref, b1_ref = refs[1 + R:6 + R]
        out_ref = refs[6 + R]

        adj_c = [a_ref[0, 0].astype(_CD) for a_ref in adj_refs]
        rel_c = rel_ref[0].astype(_CD)

        h = x_ref[0]
        for w_ref, b_ref in ((w0_ref, b0_ref), (w1_ref, b1_ref)):
            din = h.shape[1]
            K = din + L
            h_c = h.astype(_CD)
            acc = b_ref[...]
            for r in range(R):
                wx_r = w_ref[:, r * K:r * K + din].astype(_CD)
                wrel_r = w_ref[:, r * K + din:(r + 1) * K].astype(_CD)
                y = jax.lax.dot_general(h_c, wx_r, _NT,
                                        preferred_element_type=jnp.float32)
                relp = jax.lax.dot_general(rel_c[r:r + 1], wrel_r, _NT,
                                           preferred_element_type=jnp.float32)
                y = (y + relp).astype(_CD)
                acc = acc + jnp.dot(adj_c[r], y,
                                    preferred_element_type=jnp.float32)
            h = jnp.maximum(acc, 0.0)
        out_ref[0] = h
    return body


def kernel(node_features, relation_features, adj, w0, b0, w1, b1):
    B, N, Din = node_features.shape
    _, R, L = relation_features.shape
    D0, D1 = w0.shape[0], w1.shape[0]

    adj_specs = [
        pl.BlockSpec((1, 1, N, N), (lambda b, rr=r: (b, rr, 0, 0)))
        for r in range(R)
    ]
    return pl.pallas_call(
        _make_body(R, L),
        out_shape=jax.ShapeDtypeStruct((B, N, D1), node_features.dtype),
        grid=(B,),
        in_specs=[pl.BlockSpec((1, N, Din), lambda b: (b, 0, 0))] + adj_specs + [
            pl.BlockSpec((1, R, L), lambda b: (b, 0, 0)),
            pl.BlockSpec(w0.shape, lambda b: (0, 0)),
            pl.BlockSpec((1, D0), lambda b: (0, 0)),
            pl.BlockSpec(w1.shape, lambda b: (0, 0)),
            pl.BlockSpec((1, D1), lambda b: (0, 0)),
        ],
        out_specs=pl.BlockSpec((1, N, D1), lambda b: (b, 0, 0)),
        compiler_params=pltpu.CompilerParams(
            dimension_semantics=("arbitrary",),
            vmem_limit_bytes=int((64 << 20) * 0.75)),
    )(node_features, *([adj] * R), relation_features,
      w0, b0.reshape(1, D0), w1, b1.reshape(1, D1))


# speedup vs baseline: 1.2986x; 1.2986x over previous
import jax
import jax.numpy as jnp
from jax.experimental import pallas as pl
from jax.experimental.pallas import tpu as pltpu

_CD = jnp.bfloat16
_NT = (((1,), (1,)), ((), ()))


def _make_body(R, L):
    def body(*refs):
        x_ref = refs[0]
        adj_refs = refs[1:1 + R]
        rel_ref, w0_ref, b0_ref, w1_ref, b1_ref = refs[1 + R:6 + R]
        out_ref = refs[6 + R]
        wx0_s, wx1_s, relp0_s, relp1_s = refs[7 + R:11 + R]
        b = pl.program_id(0)

        @pl.when(b == 0)
        def _prep():
            rel_c = rel_ref[...].astype(_CD)
            for w_ref, wx_s, relp_s in ((w0_ref, wx0_s, relp0_s),
                                        (w1_ref, wx1_s, relp1_s)):
                din = wx_s.shape[0]
                D = b0_ref.shape[1] if wx_s is wx0_s else b1_ref.shape[1]
                K = din + L
                for r in range(R):
                    wx_r = w_ref[:, r * K:r * K + din].astype(_CD)
                    wx_s[:, r * D:(r + 1) * D] = wx_r.T
                    wrel_r = w_ref[:, r * K + din:(r + 1) * K].astype(_CD)
                    relp_s[:, r * D:(r + 1) * D] = jax.lax.dot_general(
                        rel_c[:, r, :], wrel_r, _NT,
                        preferred_element_type=jnp.float32)

        adj_c = [a_ref[0, 0].astype(_CD) for a_ref in adj_refs]

        h = x_ref[0]
        for wx_s, relp_s, b_ref in ((wx0_s, relp0_s, b0_ref),
                                    (wx1_s, relp1_s, b1_ref)):
            D = b_ref.shape[1]
            y = jnp.dot(h.astype(_CD), wx_s[...],
                        preferred_element_type=jnp.float32)
            y = (y + relp_s[pl.ds(b, 1), :]).astype(_CD)
            acc = jnp.dot(adj_c[0], y[:, :D],
                          preferred_element_type=jnp.float32)
            for r in range(1, R):
                acc += jnp.dot(adj_c[r], y[:, r * D:(r + 1) * D],
                               preferred_element_type=jnp.float32)
            h = jnp.maximum(acc + b_ref[...], 0.0)
        out_ref[0] = h
    return body


def kernel(node_features, relation_features, adj, w0, b0, w1, b1):
    B, N, Din = node_features.shape
    _, R, L = relation_features.shape
    D0, D1 = w0.shape[0], w1.shape[0]

    adj_specs = [
        pl.BlockSpec((1, 1, N, N), (lambda b, rr=r: (b, rr, 0, 0)))
        for r in range(R)
    ]
    return pl.pallas_call(
        _make_body(R, L),
        out_shape=jax.ShapeDtypeStruct((B, N, D1), node_features.dtype),
        grid=(B,),
        in_specs=[pl.BlockSpec((1, N, Din), lambda b: (b, 0, 0))] + adj_specs + [
            pl.BlockSpec((B, R, L), lambda b: (0, 0, 0)),
            pl.BlockSpec(w0.shape, lambda b: (0, 0)),
            pl.BlockSpec((1, D0), lambda b: (0, 0)),
            pl.BlockSpec(w1.shape, lambda b: (0, 0)),
            pl.BlockSpec((1, D1), lambda b: (0, 0)),
        ],
        out_specs=pl.BlockSpec((1, N, D1), lambda b: (b, 0, 0)),
        scratch_shapes=[
            pltpu.VMEM((Din, R * D0), _CD),
            pltpu.VMEM((D0, R * D1), _CD),
            pltpu.VMEM((B, R * D0), jnp.float32),
            pltpu.VMEM((B, R * D1), jnp.float32),
        ],
        compiler_params=pltpu.CompilerParams(
            dimension_semantics=("arbitrary",),
            vmem_limit_bytes=int((64 << 20) * 0.75)),
    )(node_features, *([adj] * R), relation_features,
      w0, b0.reshape(1, D0), w1, b1.reshape(1, D1))
